# baseline (speedup 1.0000x reference)
import jax
import jax.numpy as jnp
from jax import lax
from jax.experimental import pallas as pl
from jax.experimental.pallas import tpu as pltpu
from jax.experimental.pallas import tpu_sc as plsc

B = 1024
F = 128
A = 100
OUT_W = F + A
NUM_TILES = A
SHIFT = B % A
OH_REP = 12

NC = 2
NS = 16
NW = NC * NS


def _sc_body(x_hbm, oh_hbm, out_hbm, oh_sp, sem):
    c = lax.axis_index("c")
    s = lax.axis_index("s")
    wid = c * NS + s

    @pl.when(s < OH_REP)
    def _():
        pltpu.sync_copy(oh_hbm, oh_sp.at[pl.ds(s * A, A)])
    plsc.subcore_barrier()

    def fire_tile(a):
        row0 = a * B
        start = lax.rem(SHIFT * a, A)
        c1 = pltpu.async_copy(x_hbm, out_hbm.at[pl.ds(row0, B), pl.ds(0, F)],
                              sem)
        c2 = pltpu.async_copy(oh_sp.at[pl.ds(start, B)],
                              out_hbm.at[pl.ds(row0, B), pl.ds(F, A)], sem)
        return (c1, c2)

    copies = []
    for k in range(NUM_TILES // NW):
        copies.extend(fire_tile(wid + NW * k))
    for cp in copies:
        cp.wait()
    @pl.when(wid < NUM_TILES % NW)
    def _():
        for cp in fire_tile(wid + NW * (NUM_TILES // NW)):
            cp.wait()


@jax.jit
def _concat_sc(x, one_hot):
    mesh = plsc.VectorSubcoreMesh(core_axis_name="c", subcore_axis_name="s")
    return pl.kernel(
        _sc_body,
        out_type=jax.ShapeDtypeStruct((B * NUM_TILES, OUT_W), jnp.float32),
        mesh=mesh,
        scratch_types=[
            pltpu.VMEM_SHARED((OH_REP * A, A), jnp.float32),
            pltpu.SemaphoreType.DMA,
        ],
    )(x, one_hot)


def kernel(x, one_hot):
    concat_batch = _concat_sc(x, one_hot.astype(x.dtype))
    act = jnp.full((B, A), jnp.nan, dtype=jnp.float32)
    val = jnp.full((B, A), jnp.nan, dtype=jnp.float32)
    mask = jnp.ones((B, A), dtype=bool)
    return (concat_batch, act, val, mask)


# speedup vs baseline: 10.0919x; 10.0919x over previous
import jax
import jax.numpy as jnp
from jax import lax
from jax.experimental import pallas as pl
from jax.experimental.pallas import tpu as pltpu
from jax.experimental.pallas import tpu_sc as plsc

B = 1024
F = 128
A = 100
OUT_W = F + A
NUM_TILES = A
SHIFT = B % A
OH_REP = 12

NC = 2
NS = 16
NW = NC * NS


def _sc_body(x_hbm, oh_hbm, out_hbm, x_sp, oh_sp, sem):
    c = lax.axis_index("c")
    s = lax.axis_index("s")
    wid = c * NS + s

    rows_per_s = B // NS
    pltpu.sync_copy(x_hbm.at[pl.ds(s * rows_per_s, rows_per_s)],
                    x_sp.at[pl.ds(s * rows_per_s, rows_per_s)])
    @pl.when(s < OH_REP)
    def _():
        pltpu.sync_copy(oh_hbm, oh_sp.at[pl.ds(s * A, A)])
    plsc.subcore_barrier()

    def fire_tile(a):
        row0 = a * B
        start = lax.rem(SHIFT * a, A)
        c1 = pltpu.async_copy(x_sp, out_hbm.at[pl.ds(row0, B), pl.ds(0, F)],
                              sem)
        c2 = pltpu.async_copy(oh_sp.at[pl.ds(start, B)],
                              out_hbm.at[pl.ds(row0, B), pl.ds(F, A)], sem)
        return (c1, c2)

    copies = []
    for k in range(NUM_TILES // NW):
        copies.extend(fire_tile(wid + NW * k))
    for cp in copies:
        cp.wait()
    @pl.when(s < 2)
    def _():
        for cp in fire_tile(NW * (NUM_TILES // NW) + 2 * c + s):
            cp.wait()


@jax.jit
def _concat_sc(x, one_hot):
    mesh = plsc.VectorSubcoreMesh(core_axis_name="c", subcore_axis_name="s")
    return pl.kernel(
        _sc_body,
        out_type=jax.ShapeDtypeStruct((B * NUM_TILES, OUT_W), jnp.float32),
        mesh=mesh,
        scratch_types=[
            pltpu.VMEM_SHARED((B, F), jnp.float32),
            pltpu.VMEM_SHARED((OH_REP * A, A), jnp.float32),
            pltpu.SemaphoreType.DMA,
        ],
    )(x, one_hot)


def kernel(x, one_hot):
    concat_batch = _concat_sc(x, one_hot.astype(x.dtype))
    act = jnp.full((B, A), jnp.nan, dtype=jnp.float32)
    val = jnp.full((B, A), jnp.nan, dtype=jnp.float32)
    mask = jnp.ones((B, A), dtype=bool)
    return (concat_batch, act, val, mask)


# speedup vs baseline: 10.0955x; 1.0004x over previous
import jax
import jax.numpy as jnp
from jax import lax
from jax.experimental import pallas as pl
from jax.experimental.pallas import tpu as pltpu
from jax.experimental.pallas import tpu_sc as plsc

B = 1024
F = 128
A = 100
OUT_W = F + A
NUM_TILES = A
SHIFT = B % A
OH_REP = 12

NC = 2
NS = 16
NW = NC * NS


def _sc_body(x_hbm, oh_hbm, out_hbm, x_sp, oh_sp, sem):
    c = lax.axis_index("c")
    s = lax.axis_index("s")
    wid = c * NS + s

    rows_per_s = B // NS
    pltpu.sync_copy(x_hbm.at[pl.ds(s * rows_per_s, rows_per_s)],
                    x_sp.at[pl.ds(s * rows_per_s, rows_per_s)])
    @pl.when(s < OH_REP)
    def _():
        pltpu.sync_copy(oh_hbm, oh_sp.at[pl.ds(s * A, A)])
    plsc.subcore_barrier()

    def fire_tile(a):
        row0 = a * B
        start = lax.rem(SHIFT * a, A)
        c1 = pltpu.async_copy(x_sp, out_hbm.at[pl.ds(row0, B), pl.ds(0, F)],
                              sem)
        c2 = pltpu.async_copy(oh_sp.at[pl.ds(start, B)],
                              out_hbm.at[pl.ds(row0, B), pl.ds(F, A)], sem)
        return (c1, c2)

    copies = []
    for k in range(NUM_TILES // NW):
        copies.extend(fire_tile(wid + NW * k))
    for cp in copies:
        cp.wait()
    @pl.when(s < 2)
    def _():
        for cp in fire_tile(NW * (NUM_TILES // NW) + 2 * c + s):
            cp.wait()


@jax.jit
def _concat_sc(x, one_hot):
    mesh = plsc.VectorSubcoreMesh(core_axis_name="c", subcore_axis_name="s")
    return pl.kernel(
        _sc_body,
        out_type=jax.ShapeDtypeStruct((B * NUM_TILES, OUT_W), jnp.float32),
        mesh=mesh,
        compiler_params=pltpu.CompilerParams(use_tc_tiling_on_sc=True),
        scratch_types=[
            pltpu.VMEM_SHARED((B, F), jnp.float32),
            pltpu.VMEM_SHARED((OH_REP * A, A), jnp.float32),
            pltpu.SemaphoreType.DMA,
        ],
    )(x, one_hot)


def kernel(x, one_hot):
    concat_batch = _concat_sc(x, one_hot.astype(x.dtype))
    act = jnp.full((B, A), jnp.nan, dtype=jnp.float32)
    val = jnp.full((B, A), jnp.nan, dtype=jnp.float32)
    mask = jnp.ones((B, A), dtype=bool)
    return (concat_batch, act, val, mask)
